# baseline (speedup 1.0000x reference)
import functools

import jax
import jax.numpy as jnp
from jax import lax
from jax.experimental import pallas as pl
from jax.experimental.pallas import tpu as pltpu
from jax.experimental.pallas import tpu_sc as plsc

B = 1024
H = 32
N = 100000
K = 7
BLK = 2048
NBLK = 49
NPAD = NBLK * BLK
LAMBDA = 0.01
SCALE = float(H) ** -0.5
PEN_BASE = -4.0
PEN_STEP = 2.0 ** -18
NEG = -3.0e38


def _topk_kernel(qe_ref, qt_ref, pe_ref, pt_ref, out_ref,
                 rv_ref, rif_ref, qtx_ref, kref):
    b = pl.program_id(0)

    @pl.when(b == 0)
    def _init():
        rv_ref[...] = jnp.full((B, K), NEG, dtype=jnp.float32)
        rif_ref[...] = jnp.zeros((B, K), dtype=jnp.float32)
        minpt = jnp.min(pt_ref[...])
        qt0 = qt_ref[...]
        qtx_ref[...] = jnp.where(qt0 <= minpt, jnp.float32(jnp.inf), qt0)

    qe = qe_ref[...]
    qnorm = jnp.sqrt(jnp.sum(qe * qe, axis=1, keepdims=True))
    qn = qe / jnp.maximum(qnorm, 1e-12)
    ce = pe_ref[...]
    cnorm = jnp.sqrt(jnp.sum(ce * ce, axis=1, keepdims=True))
    cn = ce / jnp.maximum(cnorm, 1e-12)
    sims = jnp.dot(qn, cn.T, preferred_element_type=jnp.float32)
    pt = pt_ref[pl.ds(b, 1), :]
    qt = qt_ref[...]
    tw = jnp.exp(-LAMBDA * jnp.abs(qt - pt))
    sims = sims * tw
    gidxf_row = (jax.lax.broadcasted_iota(jnp.int32, (1, BLK), 1)
                 .astype(jnp.float32) + jnp.float32(b * BLK))
    pen_row = PEN_BASE - gidxf_row * PEN_STEP
    keys = jnp.where(pt < qtx_ref[...], sims, pen_row)

    rmin = rv_ref[:, K - 1:K]
    cntf = jnp.sum((keys > rmin).astype(jnp.float32), axis=1, keepdims=True)
    maxcnt = jnp.max(cntf)
    kref[...] = keys

    for j in range(K):
        @pl.when(jnp.float32(j) < maxcnt)
        def _extract(j=j):
            k = kref[...]
            mx = jnp.max(k, axis=1, keepdims=True)
            eq = k == mx
            amf = jnp.min(jnp.where(eq, gidxf_row, jnp.float32(3e38)),
                          axis=1, keepdims=True)
            kref[...] = jnp.where(eq, NEG, k)
            rv = rv_ref[...]
            ri = rif_ref[...]
            shift_v = jnp.concatenate(
                [jnp.full((B, 1), jnp.inf, jnp.float32), rv[:, :K - 1]], axis=1)
            shift_i = jnp.concatenate([ri[:, :1], ri[:, :K - 1]], axis=1)
            ge = rv >= mx
            gesh = shift_v >= mx
            rv_ref[...] = jnp.where(ge, rv, jnp.where(gesh, mx, shift_v))
            rif_ref[...] = jnp.where(ge, ri, jnp.where(gesh, amf, shift_i))

    @pl.when(b == NBLK - 1)
    def _emit():
        rif = rif_ref[...]
        rif = jnp.minimum(jnp.maximum(rif, 0.0), jnp.float32(N - 1))
        out_ref[...] = (rif + 0.5).astype(jnp.int32)


def _topk_call(qe, qt, pe, pt):
    return pl.pallas_call(
        _topk_kernel,
        grid=(NBLK,),
        in_specs=[
            pl.BlockSpec((B, H), lambda b: (0, 0)),
            pl.BlockSpec((B, 1), lambda b: (0, 0)),
            pl.BlockSpec((BLK, H), lambda b: (b, 0)),
            pl.BlockSpec((NBLK, BLK), lambda b: (0, 0)),
        ],
        out_specs=pl.BlockSpec((B, K), lambda b: (0, 0)),
        out_shape=jax.ShapeDtypeStruct((B, K), jnp.int32),
        scratch_shapes=[
            pltpu.VMEM((B, K), jnp.float32),
            pltpu.VMEM((B, K), jnp.float32),
            pltpu.VMEM((B, 1), jnp.float32),
            pltpu.VMEM((B, BLK), jnp.float32),
        ],
    )(qe, qt, pe, pt)


def _sc_gather(table, idx):
    info = plsc.get_sparse_core_info()
    nw = info.num_cores * info.num_subcores
    m = idx.shape[0]
    per_w = m // nw
    mesh = plsc.VectorSubcoreMesh(core_axis_name="c", subcore_axis_name="s")

    @functools.partial(
        pl.kernel, mesh=mesh,
        out_type=jax.ShapeDtypeStruct((m, H), jnp.float32),
        compiler_params=pltpu.CompilerParams(use_tc_tiling_on_sc=False),
        scratch_types=[
            pltpu.VMEM((per_w,), jnp.int32),
            pltpu.VMEM((per_w, H), jnp.float32),
            pltpu.SemaphoreType.DMA,
        ],
    )
    def gk(table_hbm, idx_hbm, out_hbm, idx_v, rows_v, sem):
        wid = lax.axis_index("s") * info.num_cores + lax.axis_index("c")
        base = wid * per_w
        pltpu.sync_copy(idx_hbm.at[pl.ds(base, per_w)], idx_v)
        pltpu.async_copy(table_hbm.at[idx_v], rows_v, sem).wait()
        pltpu.sync_copy(rows_v, out_hbm.at[pl.ds(base, per_w)])

    return gk(table, idx)


def _fusion_kernel(qe_ref, retr_ref, out_ref):
    q = qe_ref[...]
    retr = retr_ref[...]
    cols = [retr[:, j * H:(j + 1) * H] for j in range(K)]
    s = [jnp.sum(q * c, axis=1, keepdims=True) * SCALE for c in cols]
    S = jnp.concatenate(s, axis=1)
    m = jnp.max(S, axis=1, keepdims=True)
    e = jnp.exp(S - m)
    w = e / jnp.sum(e, axis=1, keepdims=True)
    fused = w[:, 0:1] * cols[0]
    for j in range(1, K):
        fused = fused + w[:, j:j + 1] * cols[j]
    qn2 = jnp.sqrt(jnp.sum(q * q, axis=1, keepdims=True))
    fn2 = jnp.sqrt(jnp.sum(fused * fused, axis=1, keepdims=True))
    dot = jnp.sum(q * fused, axis=1, keepdims=True)
    cos = dot / jnp.maximum(qn2 * fn2, 1e-8)
    d = q - fused
    l2 = jnp.sqrt(jnp.sum(d * d, axis=1, keepdims=True))
    out_ref[...] = 0.5 * (1.0 - cos) + 0.5 * l2


def _fusion_call(qe, retr):
    return pl.pallas_call(
        _fusion_kernel,
        in_specs=[
            pl.BlockSpec((B, H), lambda: (0, 0)),
            pl.BlockSpec((B, K * H), lambda: (0, 0)),
        ],
        out_specs=pl.BlockSpec((B, 1), lambda: (0, 0)),
        out_shape=jax.ShapeDtypeStruct((B, 1), jnp.float32),
    )(qe, retr)


@jax.jit
def _run(query_emb, query_time, pool_emb, pool_time):
    pe = jnp.pad(pool_emb, ((0, NPAD - N), (0, 0)))
    pt = jnp.pad(pool_time, (0, NPAD - N),
                 constant_values=jnp.inf).reshape(NBLK, BLK)
    qt = query_time.reshape(B, 1)
    ri = _topk_call(query_emb, qt, pe, pt)
    retr = _sc_gather(pool_emb, ri.reshape(B * K))
    out = _fusion_call(query_emb, retr.reshape(B, K * H))
    return out.reshape(B)


def kernel(query_emb, query_time, pool_emb, pool_time):
    return _run(query_emb, query_time, pool_emb, pool_time)


# speedup vs baseline: 1.0182x; 1.0182x over previous
import functools

import jax
import jax.numpy as jnp
from jax import lax
from jax.experimental import pallas as pl
from jax.experimental.pallas import tpu as pltpu
from jax.experimental.pallas import tpu_sc as plsc

B = 1024
H = 32
N = 100000
K = 7
BLK = 1024
NBLK = 98
NPAD = NBLK * BLK
LAMBDA = 0.01
SCALE = float(H) ** -0.5
PEN_BASE = -4.0
PEN_STEP = 2.0 ** -18
NEG = -3.0e38


def _topk_kernel(qe_ref, qt_ref, pe_ref, pt_ref, out_ref,
                 rv_ref, rif_ref, qtx_ref, kref):
    b = pl.program_id(0)

    @pl.when(b == 0)
    def _init():
        rv_ref[...] = jnp.full((B, K), NEG, dtype=jnp.float32)
        rif_ref[...] = jnp.zeros((B, K), dtype=jnp.float32)
        minpt = jnp.min(pt_ref[...])
        qt0 = qt_ref[...]
        qtx_ref[...] = jnp.where(qt0 <= minpt, jnp.float32(jnp.inf), qt0)

    qe = qe_ref[...]
    qnorm = jnp.sqrt(jnp.sum(qe * qe, axis=1, keepdims=True))
    qn = qe / jnp.maximum(qnorm, 1e-12)
    ce = pe_ref[...]
    cnorm = jnp.sqrt(jnp.sum(ce * ce, axis=1, keepdims=True))
    cn = ce / jnp.maximum(cnorm, 1e-12)
    sims = jnp.dot(qn, cn.T, preferred_element_type=jnp.float32)
    pt = pt_ref[pl.ds(b, 1), :]
    qt = qt_ref[...]
    tw = jnp.exp(-LAMBDA * jnp.abs(qt - pt))
    sims = sims * tw
    gidxf_row = (jax.lax.broadcasted_iota(jnp.int32, (1, BLK), 1)
                 .astype(jnp.float32) + jnp.float32(b * BLK))
    pen_row = PEN_BASE - gidxf_row * PEN_STEP
    keys = jnp.where(pt < qtx_ref[...], sims, pen_row)

    rmin = rv_ref[:, K - 1:K]
    cntf = jnp.sum((keys > rmin).astype(jnp.float32), axis=1, keepdims=True)
    maxcnt = jnp.max(cntf)
    kref[...] = keys

    for j in range(K):
        @pl.when(jnp.float32(j) < maxcnt)
        def _extract(j=j):
            k = kref[...]
            mx = jnp.max(k, axis=1, keepdims=True)
            eq = k == mx
            amf = jnp.min(jnp.where(eq, gidxf_row, jnp.float32(3e38)),
                          axis=1, keepdims=True)
            kref[...] = jnp.where(eq, NEG, k)
            rv = rv_ref[...]
            ri = rif_ref[...]
            shift_v = jnp.concatenate(
                [jnp.full((B, 1), jnp.inf, jnp.float32), rv[:, :K - 1]], axis=1)
            shift_i = jnp.concatenate([ri[:, :1], ri[:, :K - 1]], axis=1)
            ge = rv >= mx
            gesh = shift_v >= mx
            rv_ref[...] = jnp.where(ge, rv, jnp.where(gesh, mx, shift_v))
            rif_ref[...] = jnp.where(ge, ri, jnp.where(gesh, amf, shift_i))

    @pl.when(b == NBLK - 1)
    def _emit():
        rif = rif_ref[...]
        rif = jnp.minimum(jnp.maximum(rif, 0.0), jnp.float32(N - 1))
        out_ref[...] = (rif + 0.5).astype(jnp.int32)


def _topk_call(qe, qt, pe, pt):
    return pl.pallas_call(
        _topk_kernel,
        grid=(NBLK,),
        in_specs=[
            pl.BlockSpec((B, H), lambda b: (0, 0)),
            pl.BlockSpec((B, 1), lambda b: (0, 0)),
            pl.BlockSpec((BLK, H), lambda b: (b, 0)),
            pl.BlockSpec((NBLK, BLK), lambda b: (0, 0)),
        ],
        out_specs=pl.BlockSpec((B, K), lambda b: (0, 0)),
        out_shape=jax.ShapeDtypeStruct((B, K), jnp.int32),
        scratch_shapes=[
            pltpu.VMEM((B, K), jnp.float32),
            pltpu.VMEM((B, K), jnp.float32),
            pltpu.VMEM((B, 1), jnp.float32),
            pltpu.VMEM((B, BLK), jnp.float32),
        ],
    )(qe, qt, pe, pt)


def _sc_gather(table, idx):
    info = plsc.get_sparse_core_info()
    nw = info.num_cores * info.num_subcores
    m = idx.shape[0]
    per_w = m // nw
    mesh = plsc.VectorSubcoreMesh(core_axis_name="c", subcore_axis_name="s")

    @functools.partial(
        pl.kernel, mesh=mesh,
        out_type=jax.ShapeDtypeStruct((m, H), jnp.float32),
        compiler_params=pltpu.CompilerParams(use_tc_tiling_on_sc=False),
        scratch_types=[
            pltpu.VMEM((per_w,), jnp.int32),
            pltpu.VMEM((per_w, H), jnp.float32),
            pltpu.SemaphoreType.DMA,
        ],
    )
    def gk(table_hbm, idx_hbm, out_hbm, idx_v, rows_v, sem):
        wid = lax.axis_index("s") * info.num_cores + lax.axis_index("c")
        base = wid * per_w
        pltpu.sync_copy(idx_hbm.at[pl.ds(base, per_w)], idx_v)
        pltpu.async_copy(table_hbm.at[idx_v], rows_v, sem).wait()
        pltpu.sync_copy(rows_v, out_hbm.at[pl.ds(base, per_w)])

    return gk(table, idx)


def _fusion_kernel(qe_ref, retr_ref, out_ref):
    q = qe_ref[...]
    retr = retr_ref[...]
    cols = [retr[:, j * H:(j + 1) * H] for j in range(K)]
    s = [jnp.sum(q * c, axis=1, keepdims=True) * SCALE for c in cols]
    S = jnp.concatenate(s, axis=1)
    m = jnp.max(S, axis=1, keepdims=True)
    e = jnp.exp(S - m)
    w = e / jnp.sum(e, axis=1, keepdims=True)
    fused = w[:, 0:1] * cols[0]
    for j in range(1, K):
        fused = fused + w[:, j:j + 1] * cols[j]
    qn2 = jnp.sqrt(jnp.sum(q * q, axis=1, keepdims=True))
    fn2 = jnp.sqrt(jnp.sum(fused * fused, axis=1, keepdims=True))
    dot = jnp.sum(q * fused, axis=1, keepdims=True)
    cos = dot / jnp.maximum(qn2 * fn2, 1e-8)
    d = q - fused
    l2 = jnp.sqrt(jnp.sum(d * d, axis=1, keepdims=True))
    out_ref[...] = 0.5 * (1.0 - cos) + 0.5 * l2


def _fusion_call(qe, retr):
    return pl.pallas_call(
        _fusion_kernel,
        in_specs=[
            pl.BlockSpec((B, H), lambda: (0, 0)),
            pl.BlockSpec((B, K * H), lambda: (0, 0)),
        ],
        out_specs=pl.BlockSpec((B, 1), lambda: (0, 0)),
        out_shape=jax.ShapeDtypeStruct((B, 1), jnp.float32),
    )(qe, retr)


@jax.jit
def _run(query_emb, query_time, pool_emb, pool_time):
    pe = jnp.pad(pool_emb, ((0, NPAD - N), (0, 0)))
    pt = jnp.pad(pool_time, (0, NPAD - N),
                 constant_values=jnp.inf).reshape(NBLK, BLK)
    qt = query_time.reshape(B, 1)
    ri = _topk_call(query_emb, qt, pe, pt)
    retr = _sc_gather(pool_emb, ri.reshape(B * K))
    out = _fusion_call(query_emb, retr.reshape(B, K * H))
    return out.reshape(B)


def kernel(query_emb, query_time, pool_emb, pool_time):
    return _run(query_emb, query_time, pool_emb, pool_time)
